# baseline (speedup 1.0000x reference)
import functools

import jax
import jax.numpy as jnp
from jax import lax
from jax.experimental import pallas as pl
from jax.experimental.pallas import tpu as pltpu
from jax.experimental.pallas import tpu_sc as plsc

NUM_CLASSES = 12
NC = 2
NS = 16
L = 16
NW = NC * NS


@functools.lru_cache(maxsize=None)
def _make_planes(total: int):
    assert total % NW == 0
    per_tile = total // NW
    chunk = 3200
    while per_tile % chunk or (per_tile // chunk) % 2 or per_tile // chunk < 4:
        chunk //= 2
    assert chunk % L == 0
    n_chunks = per_tile // chunk

    mesh = plsc.VectorSubcoreMesh(core_axis_name="c", subcore_axis_name="s")

    @functools.partial(
        pl.kernel,
        out_type=jax.ShapeDtypeStruct((NUM_CLASSES, total), jnp.float32),
        mesh=mesh,
        compiler_params=pltpu.CompilerParams(needs_layout_passes=False),
        scratch_types=[
            pltpu.VMEM((chunk,), jnp.float32),
            pltpu.VMEM((chunk,), jnp.float32),
            pltpu.VMEM((NUM_CLASSES, chunk), jnp.float32),
            pltpu.VMEM((NUM_CLASSES, chunk), jnp.float32),
            pltpu.SemaphoreType.DMA,
            pltpu.SemaphoreType.DMA,
            pltpu.SemaphoreType.DMA,
            pltpu.SemaphoreType.DMA,
        ],
    )
    def planes(x_hbm, out_hbm, x0, x1, buf0, buf1, so0, so1, si0, si1):
        wid = lax.axis_index("s") * NC + lax.axis_index("c")
        tile_base = wid * per_tile
        xs = (x0, x1)
        bufs = (buf0, buf1)
        osems = (so0, so1)
        isems = (si0, si1)
        ones = jnp.ones((L,), jnp.float32)
        zeros = jnp.zeros((L,), jnp.float32)

        def start_in(b, k):
            goff = tile_base + k * chunk
            pltpu.async_copy(x_hbm.at[pl.ds(goff, chunk)], xs[b], isems[b])

        def wait_in(b):
            pltpu.make_async_copy(
                x_hbm.at[pl.ds(0, chunk)], xs[b], isems[b]
            ).wait()

        def compute_and_send(b, k):
            goff = tile_base + k * chunk

            @pl.loop(0, chunk // (2 * L))
            def _(t):
                for u in range(2):
                    off = (2 * t + u) * L
                    xv = xs[b][pl.ds(off, L)]
                    for c in range(NUM_CLASSES):
                        ov = jnp.where(xv == float(c), ones, zeros)
                        bufs[b][c, pl.ds(off, L)] = ov

            pltpu.async_copy(
                bufs[b],
                out_hbm.at[:, pl.ds(goff, chunk)],
                osems[b],
            )

        def drain(b):
            pltpu.make_async_copy(
                bufs[b],
                out_hbm.at[:, pl.ds(0, chunk)],
                osems[b],
            ).wait()

        start_in(0, 0)
        start_in(1, 1)
        for b in range(2):
            wait_in(b)
            compute_and_send(b, b)
            if n_chunks > 2:
                start_in(b, b + 2)

        @pl.loop(2, n_chunks - 2, step=2)
        def _(k):
            for b in range(2):
                drain(b)
                wait_in(b)
                compute_and_send(b, k + b)
                start_in(b, k + b + 2)

        for b in range(2):
            drain(b)
            wait_in(b)
            compute_and_send(b, n_chunks - 2 + b)
        drain(0)
        drain(1)

    return planes


def kernel(x):
    rows, cols = x.shape
    total = x.size
    xt_flat = x.T.reshape(total)
    out_2d = _make_planes(total)(xt_flat)
    out_t = out_2d.reshape(NUM_CLASSES, cols, rows)
    return out_t.transpose(2, 1, 0)


# speedup vs baseline: 1.3935x; 1.3935x over previous
import functools

import jax
import jax.numpy as jnp
from jax import lax
from jax.experimental import pallas as pl
from jax.experimental.pallas import tpu as pltpu
from jax.experimental.pallas import tpu_sc as plsc

NUM_CLASSES = 12
NC = 2
NS = 16
L = 16
NW = NC * NS


@functools.lru_cache(maxsize=None)
def _make_planes(total: int):
    assert total % NW == 0
    per_tile = total // NW
    chunk = 3200
    while per_tile % chunk or (per_tile // chunk) % 2 or per_tile // chunk < 4:
        chunk //= 2
    assert chunk % L == 0
    n_chunks = per_tile // chunk
    out_c = chunk * NUM_CLASSES

    mesh = plsc.VectorSubcoreMesh(core_axis_name="c", subcore_axis_name="s")

    @functools.partial(
        pl.kernel,
        out_type=jax.ShapeDtypeStruct((total * NUM_CLASSES,), jnp.float32),
        mesh=mesh,
        compiler_params=pltpu.CompilerParams(needs_layout_passes=False),
        scratch_types=[
            pltpu.VMEM((chunk,), jnp.float32),
            pltpu.VMEM((chunk,), jnp.float32),
            pltpu.VMEM((out_c,), jnp.float32),
            pltpu.VMEM((out_c,), jnp.float32),
            pltpu.SemaphoreType.DMA,
            pltpu.SemaphoreType.DMA,
            pltpu.SemaphoreType.DMA,
            pltpu.SemaphoreType.DMA,
        ],
    )
    def planes(x_hbm, out_hbm, x0, x1, buf0, buf1, so0, so1, si0, si1):
        wid = lax.axis_index("s") * NC + lax.axis_index("c")
        tile_base = wid * per_tile
        xs = (x0, x1)
        bufs = (buf0, buf1)
        osems = (so0, so1)
        isems = (si0, si1)
        ones = jnp.ones((L,), jnp.float32)
        zeros = jnp.zeros((L,), jnp.float32)

        def start_in(b, k):
            goff = tile_base + k * chunk
            pltpu.async_copy(x_hbm.at[pl.ds(goff, chunk)], xs[b], isems[b])

        def wait_in(b):
            pltpu.make_async_copy(
                x_hbm.at[pl.ds(0, chunk)], xs[b], isems[b]
            ).wait()

        def compute_and_send(b, k):
            goff = tile_base + k * chunk

            @pl.loop(0, chunk // (2 * L))
            def _(t):
                for u in range(2):
                    off = (2 * t + u) * L
                    xv = xs[b][pl.ds(off, L)]
                    for c in range(NUM_CLASSES):
                        ov = jnp.where(xv == float(c), ones, zeros)
                        bufs[b][pl.ds(c * chunk + off, L)] = ov

            for c in range(NUM_CLASSES):
                pltpu.async_copy(
                    bufs[b].at[pl.ds(c * chunk, chunk)],
                    out_hbm.at[pl.ds(c * total + goff, chunk)],
                    osems[b],
                )

        def drain(b):
            for c in range(NUM_CLASSES):
                pltpu.make_async_copy(
                    bufs[b].at[pl.ds(c * chunk, chunk)],
                    out_hbm.at[pl.ds(0, chunk)],
                    osems[b],
                ).wait()

        start_in(0, 0)
        start_in(1, 1)
        for b in range(2):
            wait_in(b)
            compute_and_send(b, b)
            if n_chunks > 2:
                start_in(b, b + 2)

        @pl.loop(2, n_chunks - 2, step=2)
        def _(k):
            for b in range(2):
                drain(b)
                wait_in(b)
                compute_and_send(b, k + b)
                start_in(b, k + b + 2)

        for b in range(2):
            drain(b)
            wait_in(b)
            compute_and_send(b, n_chunks - 2 + b)
        drain(0)
        drain(1)

    return planes


def kernel(x):
    rows, cols = x.shape
    total = x.size
    xt_flat = x.T.reshape(total)
    out_flat = _make_planes(total)(xt_flat)
    out_t = out_flat.reshape(NUM_CLASSES, cols, rows)
    return out_t.transpose(2, 1, 0)
